# baseline (speedup 1.0000x reference)
import functools

import jax
import jax.numpy as jnp
from jax import lax
from jax.experimental import pallas as pl
from jax.experimental.pallas import tpu as pltpu
from jax.experimental.pallas import tpu_sc as plsc

B, H, W, T, BS, D = 8, 14, 14, 12, 3, 768
N = D // 4
HW = H * W
SITES = B * HW
U_W = 3 * N
NWORKERS = 32
SPW = HW // 4
LN10K = 9.210340371976184


def _tables_body(gsd_ref, months_ref, ch_ref, pos_ref, mt_ref, u_ref, se_ref):
    months = months_ref[...]
    mk3 = lax.broadcast_in_dim(months, (B, T, N), (0, 1))
    memb = jnp.zeros((B, T, N), jnp.float32)
    for k in range(13):
        row = lax.broadcast_in_dim(mt_ref[k, :], (B, T, N), (2,))
        memb = memb + jnp.where(mk3 == k, row, 0.0)
    chb = lax.broadcast_in_dim(ch_ref[...], (B, T, BS, N), (2, 3))
    posb = lax.broadcast_in_dim(pos_ref[...][:T], (B, T, BS, N), (1, 3))
    membb = lax.broadcast_in_dim(memb, (B, T, BS, N), (0, 1, 3))
    u_ref[...] = jnp.concatenate([chb, posb, membb], axis=-1)

    gsd = gsd_ref[0, 0]
    qq = lax.broadcasted_iota(jnp.int32, (4, SPW, N // 4), 0)
    kk = lax.broadcasted_iota(jnp.int32, (4, SPW, N // 4), 1)
    site = qq * SPW + kk
    ki = lax.broadcasted_iota(jnp.int32, (4, SPW, N // 4), 2).astype(jnp.float32)
    omega = jnp.exp(ki * (-LN10K / (N // 4)))
    py = (site // W).astype(jnp.float32) * gsd
    px = (site % W).astype(jnp.float32) * gsd
    oy = py * omega
    ox = px * omega
    se_ref[...] = jnp.concatenate(
        [jnp.sin(oy), jnp.cos(oy), jnp.sin(ox), jnp.cos(ox)], axis=-1)


def _build_tables(gsd, months, channel_embed, pos_embed, month_tab):
    return pl.pallas_call(
        _tables_body,
        out_shape=(
            jax.ShapeDtypeStruct((B, T, BS, U_W), jnp.float32),
            jax.ShapeDtypeStruct((4, SPW, N), jnp.float32),
        ),
        in_specs=[
            pl.BlockSpec(memory_space=pltpu.SMEM),
            pl.BlockSpec(memory_space=pltpu.VMEM),
            pl.BlockSpec(memory_space=pltpu.VMEM),
            pl.BlockSpec(memory_space=pltpu.VMEM),
            pl.BlockSpec(memory_space=pltpu.VMEM),
        ],
    )(gsd, months, channel_embed, pos_embed, month_tab)


def _sc_add_body(tok_hbm, u_hbm, se_hbm, out_hbm, u_v, se_v, tok_v):
    c = lax.axis_index("c")
    s = lax.axis_index("s")
    wid = c * 16 + s
    bidx = wid // 4
    q = wid % 4
    pltpu.sync_copy(u_hbm.at[bidx], u_v)
    pltpu.sync_copy(se_hbm.at[q], se_v)

    def unit_body(k, carry):
        hwsite = q * SPW + k
        hh = hwsite // W
        ww = hwsite % W
        pltpu.sync_copy(tok_hbm.at[bidx, hh, ww], tok_v)
        sev = [se_v[k, pl.ds(i * 16, 16)] for i in range(N // 16)]

        def row_body(j, c2):
            for bsi in range(BS):
                for i in range(U_W // 16):
                    plsc.addupdate(tok_v.at[j, bsi, pl.ds(i * 16, 16)],
                                   u_v[j, bsi, pl.ds(i * 16, 16)])
                for i in range(N // 16):
                    plsc.addupdate(tok_v.at[j, bsi, pl.ds(U_W + i * 16, 16)],
                                   sev[i])
            return c2

        lax.fori_loop(0, T, row_body, 0)
        pltpu.sync_copy(tok_v, out_hbm.at[bidx, hh, ww])
        return carry

    lax.fori_loop(0, SPW, unit_body, 0)


@functools.cache
def _sc_add():
    return functools.partial(
        pl.kernel,
        out_type=jax.ShapeDtypeStruct((B, H, W, T, BS, D), jnp.float32),
        mesh=plsc.VectorSubcoreMesh(core_axis_name="c", subcore_axis_name="s",
                                    num_cores=2, num_subcores=16),
        compiler_params=pltpu.CompilerParams(use_tc_tiling_on_sc=True),
        scratch_types=[
            pltpu.VMEM((T, BS, U_W), jnp.float32),
            pltpu.VMEM((SPW, N), jnp.float32),
            pltpu.VMEM((T, BS, D), jnp.float32),
        ],
    )(_sc_add_body)


def kernel(modality_tokens, timestamps, channel_embed, pos_embed, month_tab,
           patch_size, input_res):
    gsd = (jnp.float32(input_res) * jnp.float32(patch_size) / 10.0).reshape(1, 1)
    months = timestamps[:, :, 1].astype(jnp.int32)
    u, se = _build_tables(gsd, months, channel_embed, pos_embed, month_tab)
    return _sc_add()(modality_tokens, u, se)


# speedup vs baseline: 1.2569x; 1.2569x over previous
import functools

import jax
import jax.numpy as jnp
from jax import lax
from jax.experimental import pallas as pl
from jax.experimental.pallas import tpu as pltpu
from jax.experimental.pallas import tpu_sc as plsc

B, H, W, T, BS, D = 8, 14, 14, 12, 3, 768
N = D // 4
HW = H * W
TBS = T * BS
U_W = 3 * N
NWORKERS = 32
LN10K = 9.210340371976184


def _tables_body(gsd_ref, months_ref, ch_ref, pos_ref, mt_ref, u_ref, se_ref):
    months = months_ref[...]
    mk3 = lax.broadcast_in_dim(months, (T, B, N), (0, 1))
    memb = jnp.zeros((T, B, N), jnp.float32)
    for k in range(13):
        row = lax.broadcast_in_dim(mt_ref[k, :], (T, B, N), (2,))
        memb = memb + jnp.where(mk3 == k, row, 0.0)
    chb = lax.broadcast_in_dim(ch_ref[...], (T, BS, B, N), (1, 3))
    posb = lax.broadcast_in_dim(pos_ref[...][:T], (T, BS, B, N), (0, 3))
    membb = lax.broadcast_in_dim(memb, (T, BS, B, N), (0, 2, 3))
    u_ref[...] = jnp.concatenate([chb, posb, membb], axis=-1)

    gsd = gsd_ref[0, 0]
    ri = lax.broadcasted_iota(jnp.int32, (HW, N // 4), 0)
    ki = lax.broadcasted_iota(jnp.int32, (HW, N // 4), 1).astype(jnp.float32)
    omega = jnp.exp(ki * (-LN10K / (N // 4)))
    py = (ri // W).astype(jnp.float32) * gsd
    px = (ri % W).astype(jnp.float32) * gsd
    oy = py * omega
    ox = px * omega
    se_ref[...] = jnp.concatenate(
        [jnp.sin(oy), jnp.cos(oy), jnp.sin(ox), jnp.cos(ox)], axis=-1)


def _build_tables(gsd, months_t, channel_embed, pos_embed, month_tab):
    return pl.pallas_call(
        _tables_body,
        out_shape=(
            jax.ShapeDtypeStruct((T, BS, B, U_W), jnp.float32),
            jax.ShapeDtypeStruct((HW, N), jnp.float32),
        ),
        in_specs=[
            pl.BlockSpec(memory_space=pltpu.SMEM),
            pl.BlockSpec(memory_space=pltpu.VMEM),
            pl.BlockSpec(memory_space=pltpu.VMEM),
            pl.BlockSpec(memory_space=pltpu.VMEM),
            pl.BlockSpec(memory_space=pltpu.VMEM),
        ],
    )(gsd, months_t, channel_embed, pos_embed, month_tab)


def _sc_add_body(tok_hbm, u_hbm, se_hbm, out_hbm, u_v, se_v, tok_v):
    c = lax.axis_index("c")
    s = lax.axis_index("s")
    wid = c * 16 + s
    lo = wid * 6 + jnp.minimum(wid, 4)
    cnt = jnp.where(wid < 4, 7, 6)
    pltpu.sync_copy(se_hbm, se_v)

    def slot_body(j36, carry):
        tt = j36 // BS
        bsi = j36 % BS
        pltpu.sync_copy(u_hbm.at[tt, bsi], u_v)

        def site_body(si, c2):
            site = lo + si
            hh = site // W
            ww = site % W
            pltpu.sync_copy(tok_hbm.at[hh, ww, tt, bsi], tok_v)
            sev = [se_v[site, pl.ds(i * 16, 16)] for i in range(N // 16)]
            for r in range(B):
                for i in range(U_W // 16):
                    plsc.addupdate(tok_v.at[r, pl.ds(i * 16, 16)],
                                   u_v[r, pl.ds(i * 16, 16)])
                for i in range(N // 16):
                    plsc.addupdate(tok_v.at[r, pl.ds(U_W + i * 16, 16)],
                                   sev[i])
            pltpu.sync_copy(tok_v, out_hbm.at[hh, ww, tt, bsi])
            return c2

        lax.fori_loop(0, cnt, site_body, 0)
        return carry

    lax.fori_loop(0, TBS, slot_body, 0)


@functools.cache
def _sc_add():
    return functools.partial(
        pl.kernel,
        out_type=jax.ShapeDtypeStruct((H, W, T, BS, B, D), jnp.float32),
        mesh=plsc.VectorSubcoreMesh(core_axis_name="c", subcore_axis_name="s",
                                    num_cores=2, num_subcores=16),
        scratch_types=[
            pltpu.VMEM((B, U_W), jnp.float32),
            pltpu.VMEM((HW, N), jnp.float32),
            pltpu.VMEM((B, D), jnp.float32),
        ],
    )(_sc_add_body)


def kernel(modality_tokens, timestamps, channel_embed, pos_embed, month_tab,
           patch_size, input_res):
    gsd = (jnp.float32(input_res) * jnp.float32(patch_size) / 10.0).reshape(1, 1)
    months_t = timestamps[:, :, 1].astype(jnp.int32).T
    u, se = _build_tables(gsd, months_t, channel_embed, pos_embed, month_tab)
    tok_t = jnp.transpose(modality_tokens, (1, 2, 3, 4, 0, 5))
    out_t = _sc_add()(tok_t, u, se)
    return jnp.transpose(out_t, (4, 0, 1, 2, 3, 5))


# speedup vs baseline: 3.4354x; 2.7334x over previous
import functools

import jax
import jax.numpy as jnp
from jax import lax
from jax.experimental import pallas as pl
from jax.experimental.pallas import tpu as pltpu
from jax.experimental.pallas import tpu_sc as plsc

B, H, W, T, BS, D = 8, 14, 14, 12, 3, 768
N = D // 4
HW = H * W
TBS = T * BS
U_W = 3 * N
NWORKERS = 32
LN10K = 9.210340371976184


def _tables_body(gsd_ref, months_ref, ch_ref, pos_ref, mt_ref, u_ref, se_ref):
    months = months_ref[...]
    mk3 = lax.broadcast_in_dim(months, (T, B, N), (0, 1))
    memb = jnp.zeros((T, B, N), jnp.float32)
    for k in range(13):
        row = lax.broadcast_in_dim(mt_ref[k, :], (T, B, N), (2,))
        memb = memb + jnp.where(mk3 == k, row, 0.0)
    chb = lax.broadcast_in_dim(ch_ref[...], (T, BS, B, N), (1, 3))
    posb = lax.broadcast_in_dim(pos_ref[...][:T], (T, BS, B, N), (0, 3))
    membb = lax.broadcast_in_dim(memb, (T, BS, B, N), (0, 2, 3))
    u_ref[...] = jnp.concatenate([chb, posb, membb], axis=-1)

    gsd = gsd_ref[0, 0]
    ri = lax.broadcasted_iota(jnp.int32, (HW, 1, N // 4), 0)
    ki = lax.broadcasted_iota(jnp.int32, (HW, 1, N // 4), 2).astype(jnp.float32)
    omega = jnp.exp(ki * (-LN10K / (N // 4)))
    py = (ri // W).astype(jnp.float32) * gsd
    px = (ri % W).astype(jnp.float32) * gsd
    oy = py * omega
    ox = px * omega
    se_ref[...] = jnp.concatenate(
        [jnp.sin(oy), jnp.cos(oy), jnp.sin(ox), jnp.cos(ox)], axis=-1)


def _build_tables(gsd, months_t, channel_embed, pos_embed, month_tab):
    return pl.pallas_call(
        _tables_body,
        out_shape=(
            jax.ShapeDtypeStruct((T, BS, B, U_W), jnp.float32),
            jax.ShapeDtypeStruct((HW, 1, N), jnp.float32),
        ),
        in_specs=[
            pl.BlockSpec(memory_space=pltpu.SMEM),
            pl.BlockSpec(memory_space=pltpu.VMEM),
            pl.BlockSpec(memory_space=pltpu.VMEM),
            pl.BlockSpec(memory_space=pltpu.VMEM),
            pl.BlockSpec(memory_space=pltpu.VMEM),
        ],
    )(gsd, months_t, channel_embed, pos_embed, month_tab)


def _sc_add_body(tok_hbm, u_hbm, se_hbm, out_hbm,
                 tok0, tok1, tok2, u0, u1, se_v,
                 sin0, sin1, sin2, sout0, sout1, sout2, su):
    c = lax.axis_index("c")
    s = lax.axis_index("s")
    wid = c * 16 + s
    lo = wid * 6 + jnp.minimum(wid, 4)
    cnt = jnp.where(wid < 4, 7, 6)
    nblk = T * cnt
    toks = [tok0, tok1, tok2]
    sins = [sin0, sin1, sin2]
    souts = [sout0, sout1, sout2]
    us = [u0, u1]

    def blk_src(m):
        tt = m // cnt
        site = lo + (m % cnt)
        return tt, site // W, site % W

    def start_in(m, buf, sem):
        tt, hh, ww = blk_src(m)
        pltpu.make_async_copy(tok_hbm.at[hh, ww, tt], buf, sem).start()

    for j in range(7):
        @pl.when(j < cnt)
        def _():
            pltpu.sync_copy(se_hbm.at[lo + j], se_v.at[j])
    pltpu.sync_copy(u_hbm.at[0], u0)
    pltpu.make_async_copy(u_hbm.at[1], u1, su).start()
    start_in(0, tok0, sin0)
    start_in(1, tok1, sin1)

    def compute(buf, ub, si):
        sev = [se_v[si, 0, pl.ds(i * 16, 16)] for i in range(N // 16)]

        def row_body(r, cc):
            for bsi in range(BS):
                for i in range(U_W // 16):
                    plsc.addupdate(buf.at[bsi, r, pl.ds(i * 16, 16)],
                                   ub[bsi, r, pl.ds(i * 16, 16)])
                for i in range(N // 16):
                    plsc.addupdate(buf.at[bsi, r, pl.ds(U_W + i * 16, 16)],
                                   sev[i])
            return cc

        lax.fori_loop(0, B, row_body, 0)

    def iter_body(m, carry):
        tt, hh, ww = blk_src(m)
        si = m % cnt
        @pl.when(jnp.logical_and(si == 0, tt >= 1))
        def _():
            pltpu.make_async_copy(u_hbm.at[0], u0, su).wait()
            for ub in range(2):
                @pl.when(jnp.logical_and((tt + 1) % 2 == ub, tt < T - 1))
                def _():
                    pltpu.make_async_copy(u_hbm.at[tt + 1], us[ub], su).start()

        for r3 in range(3):
            @pl.when(m % 3 == r3)
            def _():
                buf = toks[r3]
                pltpu.make_async_copy(tok_hbm.at[hh, ww, tt], buf,
                                      sins[r3]).wait()
                for ub in range(2):
                    @pl.when(tt % 2 == ub)
                    def _():
                        compute(buf, us[ub], si)
                pltpu.make_async_copy(buf, out_hbm.at[hh, ww, tt],
                                      souts[r3]).start()
                @pl.when(m + 2 < nblk)
                def _():
                    @pl.when(m >= 1)
                    def _():
                        pltpu.make_async_copy(toks[(r3 + 2) % 3], out_hbm.at[0, 0, 0],
                                              souts[(r3 + 2) % 3]).wait()
                    start_in(m + 2, toks[(r3 + 2) % 3], sins[(r3 + 2) % 3])
        return carry

    lax.fori_loop(0, nblk, iter_body, 0)
    pltpu.make_async_copy(tok0, out_hbm.at[0, 0, 0], sout0).wait()
    pltpu.make_async_copy(tok1, out_hbm.at[0, 0, 0], sout1).wait()
    pltpu.make_async_copy(tok2, out_hbm.at[0, 0, 0], sout2).wait()


@functools.cache
def _sc_add():
    return functools.partial(
        pl.kernel,
        out_type=jax.ShapeDtypeStruct((H, W, T, BS, B, D), jnp.float32),
        mesh=plsc.VectorSubcoreMesh(core_axis_name="c", subcore_axis_name="s",
                                    num_cores=2, num_subcores=16),
        scratch_types=[
            pltpu.VMEM((BS, B, D), jnp.float32),
            pltpu.VMEM((BS, B, D), jnp.float32),
            pltpu.VMEM((BS, B, D), jnp.float32),
            pltpu.VMEM((BS, B, U_W), jnp.float32),
            pltpu.VMEM((BS, B, U_W), jnp.float32),
            pltpu.VMEM((8, 1, N), jnp.float32),
            pltpu.SemaphoreType.DMA,
            pltpu.SemaphoreType.DMA,
            pltpu.SemaphoreType.DMA,
            pltpu.SemaphoreType.DMA,
            pltpu.SemaphoreType.DMA,
            pltpu.SemaphoreType.DMA,
            pltpu.SemaphoreType.DMA,
        ],
    )(_sc_add_body)


def kernel(modality_tokens, timestamps, channel_embed, pos_embed, month_tab,
           patch_size, input_res):
    gsd = (jnp.float32(input_res) * jnp.float32(patch_size) / 10.0).reshape(1, 1)
    months_t = timestamps[:, :, 1].astype(jnp.int32).T
    u, se = _build_tables(gsd, months_t, channel_embed, pos_embed, month_tab)
    tok_t = jnp.transpose(modality_tokens, (1, 2, 3, 4, 0, 5))
    out_t = _sc_add()(tok_t, u, se)
    return jnp.transpose(out_t, (4, 0, 1, 2, 3, 5))


# speedup vs baseline: 3.9302x; 1.1440x over previous
import functools

import jax
import jax.numpy as jnp
from jax import lax
from jax.experimental import pallas as pl
from jax.experimental.pallas import tpu as pltpu
from jax.experimental.pallas import tpu_sc as plsc

B, H, W, T, BS, D = 8, 14, 14, 12, 3, 768
N = D // 4
HW = H * W
TBS = T * BS
U_W = 3 * N
NWORKERS = 32
LN10K = 9.210340371976184


def _tables_body(gsd_ref, months_ref, ch_ref, pos_ref, mt_ref, u_ref, se_ref):
    months = months_ref[...]
    mk3 = lax.broadcast_in_dim(months, (T, B, N), (0, 1))
    memb = jnp.zeros((T, B, N), jnp.float32)
    for k in range(13):
        row = lax.broadcast_in_dim(mt_ref[k, :], (T, B, N), (2,))
        memb = memb + jnp.where(mk3 == k, row, 0.0)
    chb = lax.broadcast_in_dim(ch_ref[...], (T, BS, B, N), (1, 3))
    posb = lax.broadcast_in_dim(pos_ref[...][:T], (T, BS, B, N), (0, 3))
    membb = lax.broadcast_in_dim(memb, (T, BS, B, N), (0, 2, 3))
    u_ref[...] = jnp.concatenate([chb, posb, membb], axis=-1)

    gsd = gsd_ref[0, 0]
    ri = lax.broadcasted_iota(jnp.int32, (HW, 1, N // 4), 0)
    ki = lax.broadcasted_iota(jnp.int32, (HW, 1, N // 4), 2).astype(jnp.float32)
    omega = jnp.exp(ki * (-LN10K / (N // 4)))
    py = (ri // W).astype(jnp.float32) * gsd
    px = (ri % W).astype(jnp.float32) * gsd
    oy = py * omega
    ox = px * omega
    se_ref[...] = jnp.concatenate(
        [jnp.sin(oy), jnp.cos(oy), jnp.sin(ox), jnp.cos(ox)], axis=-1)


def _build_tables(gsd, months_t, channel_embed, pos_embed, month_tab):
    return pl.pallas_call(
        _tables_body,
        out_shape=(
            jax.ShapeDtypeStruct((T, BS, B, U_W), jnp.float32),
            jax.ShapeDtypeStruct((HW, 1, N), jnp.float32),
        ),
        in_specs=[
            pl.BlockSpec(memory_space=pltpu.SMEM),
            pl.BlockSpec(memory_space=pltpu.VMEM),
            pl.BlockSpec(memory_space=pltpu.VMEM),
            pl.BlockSpec(memory_space=pltpu.VMEM),
            pl.BlockSpec(memory_space=pltpu.VMEM),
        ],
    )(gsd, months_t, channel_embed, pos_embed, month_tab)


def _sc_add_body(tok_hbm, u_hbm, se_hbm, out_hbm,
                 tok0, tok1, tok2, u0, u1, seb0, seb1, seb2,
                 sin0, sin1, sin2, sout0, sout1, sout2, su):
    c = lax.axis_index("c")
    s = lax.axis_index("s")
    wid = c * 16 + s
    lo_g = wid * 73 + jnp.minimum(wid, 16)
    nblk = jnp.where(wid < 16, 74, 73)
    toks = [tok0, tok1, tok2]
    sebs = [seb0, seb1, seb2]
    sins = [sin0, sin1, sin2]
    souts = [sout0, sout1, sout2]
    us = [u0, u1]

    tt_first = lo_g // HW
    crossed = (lo_g + nblk - 1) // HW != tt_first

    def blk_src(m):
        g = lo_g + m
        tt = g // HW
        site = g % HW
        return tt, site, site // W, site % W

    def start_in(m, r3):
        tt, site, hh, ww = blk_src(m)
        pltpu.make_async_copy(tok_hbm.at[hh, ww, tt], toks[r3], sins[r3]).start()
        pltpu.make_async_copy(se_hbm.at[site], sebs[r3], sins[r3]).start()

    def wait_in(m, r3):
        tt, site, hh, ww = blk_src(m)
        pltpu.make_async_copy(tok_hbm.at[hh, ww, tt], toks[r3], sins[r3]).wait()
        pltpu.make_async_copy(se_hbm.at[site], sebs[r3], sins[r3]).wait()

    for ub in range(2):
        @pl.when(tt_first % 2 == ub)
        def _():
            pltpu.sync_copy(u_hbm.at[tt_first], us[ub])
        @pl.when(jnp.logical_and((tt_first + 1) % 2 == ub, crossed))
        def _():
            pltpu.make_async_copy(u_hbm.at[tt_first + 1], us[ub], su).start()
    start_in(0, 0)
    start_in(1, 1)

    def compute(buf, ub, seb):
        sev = [seb[0, pl.ds(i * 16, 16)] for i in range(N // 16)]

        def row_body(r, cc):
            for bsi in range(BS):
                for i in range(U_W // 16):
                    plsc.addupdate(buf.at[bsi, r, pl.ds(i * 16, 16)],
                                   ub[bsi, r, pl.ds(i * 16, 16)])
                for i in range(N // 16):
                    plsc.addupdate(buf.at[bsi, r, pl.ds(U_W + i * 16, 16)],
                                   sev[i])
            return cc

        lax.fori_loop(0, B, row_body, 0)

    def iter_body(m, carry):
        tt, site, hh, ww = blk_src(m)
        @pl.when(jnp.logical_and(site == 0, m > 0))
        def _():
            pltpu.make_async_copy(u_hbm.at[0], u0, su).wait()

        for r3 in range(3):
            @pl.when(m % 3 == r3)
            def _():
                wait_in(m, r3)
                for ub in range(2):
                    @pl.when(tt % 2 == ub)
                    def _():
                        compute(toks[r3], us[ub], sebs[r3])
                pltpu.make_async_copy(toks[r3], out_hbm.at[hh, ww, tt],
                                      souts[r3]).start()
                @pl.when(m + 2 < nblk)
                def _():
                    @pl.when(m >= 1)
                    def _():
                        pltpu.make_async_copy(toks[(r3 + 2) % 3],
                                              out_hbm.at[0, 0, 0],
                                              souts[(r3 + 2) % 3]).wait()
                    start_in(m + 2, (r3 + 2) % 3)
        return carry

    lax.fori_loop(0, nblk, iter_body, 0)
    pltpu.make_async_copy(tok0, out_hbm.at[0, 0, 0], sout0).wait()
    pltpu.make_async_copy(tok1, out_hbm.at[0, 0, 0], sout1).wait()
    pltpu.make_async_copy(tok2, out_hbm.at[0, 0, 0], sout2).wait()


@functools.cache
def _sc_add():
    return functools.partial(
        pl.kernel,
        out_type=jax.ShapeDtypeStruct((H, W, T, BS, B, D), jnp.float32),
        mesh=plsc.VectorSubcoreMesh(core_axis_name="c", subcore_axis_name="s",
                                    num_cores=2, num_subcores=16),
        scratch_types=[
            pltpu.VMEM((BS, B, D), jnp.float32),
            pltpu.VMEM((BS, B, D), jnp.float32),
            pltpu.VMEM((BS, B, D), jnp.float32),
            pltpu.VMEM((BS, B, U_W), jnp.float32),
            pltpu.VMEM((BS, B, U_W), jnp.float32),
            pltpu.VMEM((1, N), jnp.float32),
            pltpu.VMEM((1, N), jnp.float32),
            pltpu.VMEM((1, N), jnp.float32),
            pltpu.SemaphoreType.DMA,
            pltpu.SemaphoreType.DMA,
            pltpu.SemaphoreType.DMA,
            pltpu.SemaphoreType.DMA,
            pltpu.SemaphoreType.DMA,
            pltpu.SemaphoreType.DMA,
            pltpu.SemaphoreType.DMA,
        ],
    )(_sc_add_body)


def kernel(modality_tokens, timestamps, channel_embed, pos_embed, month_tab,
           patch_size, input_res):
    gsd = (jnp.float32(input_res) * jnp.float32(patch_size) / 10.0).reshape(1, 1)
    months_t = timestamps[:, :, 1].astype(jnp.int32).T
    u, se = _build_tables(gsd, months_t, channel_embed, pos_embed, month_tab)
    tok_t = jnp.transpose(modality_tokens, (1, 2, 3, 4, 0, 5))
    out_t = _sc_add()(tok_t, u, se)
    return jnp.transpose(out_t, (4, 0, 1, 2, 3, 5))
